# baseline (speedup 1.0000x reference)
import functools

import jax
import jax.numpy as jnp
from jax import lax
from jax.experimental import pallas as pl
from jax.experimental.pallas import tpu as pltpu
from jax.experimental.pallas import tpu_sc as plsc

DIM = 2048
N_EXPERTS = 64
N_GROUPS = 8
GROUP_SIZE = N_EXPERTS // N_GROUPS
TOPK_GROUPS = 4
TOP_K = 8
ROUTE_SCALE = 2.5

NEG_INF = float("-inf")

N_TILES = 32
VEC = 16


def _scores_block(x_ref, w_ref, out_ref):
    blk_tiles = out_ref.shape[0]
    tok_per_tile = out_ref.shape[2]
    logits = lax.dot_general(
        w_ref[...], x_ref[...],
        (((1,), (1,)), ((), ())),
        preferred_element_type=jnp.float32,
    )
    scores = jax.nn.sigmoid(logits)
    for j in range(blk_tiles):
        out_ref[j, :, :] = scores[:, j * tok_per_tile:(j + 1) * tok_per_tile]


def _tc_scores(x, weight, n, block_m, tok_per_tile):
    blk_tiles = block_m // tok_per_tile
    return pl.pallas_call(
        _scores_block,
        grid=(n // block_m,),
        in_specs=[
            pl.BlockSpec((block_m, DIM), lambda i: (i, 0)),
            pl.BlockSpec((N_EXPERTS, DIM), lambda i: (0, 0)),
        ],
        out_specs=pl.BlockSpec(
            (blk_tiles, N_EXPERTS, tok_per_tile), lambda i: (i, 0, 0)),
        out_shape=jax.ShapeDtypeStruct(
            (n // tok_per_tile, N_EXPERTS, tok_per_tile), jnp.float32),
    )(x, weight)


def _combine(av, ai, bv, bi):
    tb = bv > av
    return jnp.where(tb, bv, av), jnp.where(tb, bi, ai)


def _route_tile(scores_hbm, out_w_hbm, out_i_hbm, sbuf, wbuf, ibuf):
    cid = lax.axis_index("c")
    sid = lax.axis_index("s")
    wid = sid * 2 + cid
    tok_per_tile = sbuf.shape[1]

    pltpu.sync_copy(scores_hbm.at[wid], sbuf)

    lane = lax.broadcasted_iota(jnp.int32, (VEC,), 0)
    neg = jnp.full((VEC,), NEG_INF, dtype=jnp.float32)

    def batch(t, carry):
        t16 = t * VEC
        sl = pl.ds(t16, VEC)

        gms = []
        for g in range(N_GROUPS):
            m = sbuf[g * GROUP_SIZE, sl]
            for j in range(1, GROUP_SIZE):
                m = jnp.maximum(m, sbuf[g * GROUP_SIZE + j, sl])
            gms.append(m)

        ranks = [jnp.zeros((VEC,), jnp.int32) for _ in range(N_GROUPS)]
        one = jnp.ones((VEC,), jnp.int32)
        zero = jnp.zeros((VEC,), jnp.int32)
        for gp in range(N_GROUPS):
            for g in range(N_GROUPS):
                if gp == g:
                    continue
                beats = (gms[gp] >= gms[g]) if gp < g else (gms[gp] > gms[g])
                ranks[g] = ranks[g] + jnp.where(beats, one, zero)
        zf = jnp.zeros((VEC,), jnp.float32)
        pens = [jnp.where(ranks[g] < TOPK_GROUPS, zf, neg)
                for g in range(N_GROUPS)]

        econsts = [jnp.full((VEC,), e, jnp.int32) for e in range(N_EXPERTS)]
        cur = [sbuf[e, sl] + pens[e // GROUP_SIZE]
               for e in range(N_EXPERTS)]

        vals, inds = [], []
        for k in range(TOP_K):
            pairs = list(zip(cur, econsts))
            while len(pairs) > 1:
                nxt = []
                for p in range(0, len(pairs), 2):
                    av, ai = pairs[p]
                    bv, bi = pairs[p + 1]
                    nxt.append(_combine(av, ai, bv, bi))
                pairs = nxt
            bv, bi = pairs[0]
            vals.append(bv)
            inds.append(bi)
            if k + 1 < TOP_K:
                cur = [jnp.where(bi == econsts[e], neg, cur[e])
                       for e in range(N_EXPERTS)]

        tot = vals[0]
        for k in range(1, TOP_K):
            tot = tot + vals[k]
        inv = ROUTE_SCALE / tot
        for k in range(TOP_K):
            wbuf[k, sl] = vals[k] * inv
            ibuf[k, sl] = inds[k]
        return carry

    lax.fori_loop(0, tok_per_tile // VEC, batch, 0)

    pltpu.sync_copy(wbuf, out_w_hbm.at[wid])
    pltpu.sync_copy(ibuf, out_i_hbm.at[wid])


def _sc_route(scores3, n, tok_per_tile):
    n_tiles = n // tok_per_tile
    mesh = plsc.VectorSubcoreMesh(core_axis_name="c", subcore_axis_name="s")
    fn = pl.kernel(
        _route_tile,
        out_type=[
            jax.ShapeDtypeStruct((n_tiles, TOP_K, tok_per_tile), jnp.float32),
            jax.ShapeDtypeStruct((n_tiles, TOP_K, tok_per_tile), jnp.int32),
        ],
        mesh=mesh,
        scratch_types=[
            pltpu.VMEM((N_EXPERTS, tok_per_tile), jnp.float32),
            pltpu.VMEM((TOP_K, tok_per_tile), jnp.float32),
            pltpu.VMEM((TOP_K, tok_per_tile), jnp.int32),
        ],
    )
    return fn(scores3)


@functools.partial(jax.jit, static_argnames=("block_m",))
def _run(x, weight, block_m=2048):
    n = x.shape[0]
    tok_per_tile = n // N_TILES
    scores3 = _tc_scores(x, weight, n, block_m, tok_per_tile)
    w3, i3 = _sc_route(scores3, n, tok_per_tile)
    out_w = jnp.swapaxes(w3, 1, 2).reshape(n, TOP_K)
    out_i = jnp.swapaxes(i3, 1, 2).reshape(n, TOP_K)
    return out_w, out_i


def kernel(x, weight):
    return tuple(_run(x, weight))


# speedup vs baseline: 1.8016x; 1.8016x over previous
import functools

import jax
import jax.numpy as jnp
from jax import lax
from jax.experimental import pallas as pl

DIM = 2048
N_EXPERTS = 64
N_GROUPS = 8
GROUP_SIZE = N_EXPERTS // N_GROUPS
TOPK_GROUPS = 4
TOP_K = 8
ROUTE_SCALE = 2.5

NEG_INF = float("-inf")


def _router_block(x_ref, w_ref, out_w_ref, out_i_ref):
    m = x_ref.shape[0]
    logits = lax.dot_general(
        w_ref[...], x_ref[...],
        (((1,), (1,)), ((), ())),
        preferred_element_type=jnp.float32,
    )
    scores = jax.nn.sigmoid(logits)

    gmax = jnp.concatenate(
        [jnp.max(scores[g * GROUP_SIZE:(g + 1) * GROUP_SIZE, :], axis=0,
                 keepdims=True)
         for g in range(N_GROUPS)], axis=0)

    gidx = lax.broadcasted_iota(jnp.int32, (N_GROUPS, m), 0).astype(
        jnp.float32)
    rank = jnp.zeros((N_GROUPS, m), dtype=jnp.float32)
    for gp in range(N_GROUPS):
        v = gmax[gp:gp + 1, :]
        beats = (v > gmax) | ((v == gmax) & (gp < gidx))
        rank = rank + jnp.where(beats, 1.0, 0.0)

    penalty = jnp.where(rank < TOPK_GROUPS, 0.0, NEG_INF)
    penalty_lane = jnp.concatenate(
        [jnp.broadcast_to(penalty[g:g + 1, :], (GROUP_SIZE, m))
         for g in range(N_GROUPS)], axis=0)
    masked = scores + penalty_lane

    lane = lax.broadcasted_iota(jnp.int32, (N_EXPERTS, m), 0).astype(
        jnp.float32)
    cur = masked
    vals, idxs = [], []
    for _ in range(TOP_K):
        mx = jnp.max(cur, axis=0, keepdims=True)
        sel = jnp.min(jnp.where(cur == mx, lane, float(N_EXPERTS)),
                      axis=0, keepdims=True)
        vals.append(mx)
        idxs.append(sel)
        cur = jnp.where(lane == sel, NEG_INF, cur)
    vals = jnp.concatenate(vals, axis=0)
    idxs = jnp.concatenate(idxs, axis=0)

    wsum = jnp.sum(vals, axis=0, keepdims=True)
    out_w_ref[...] = vals / wsum * ROUTE_SCALE
    out_i_ref[...] = idxs.astype(jnp.int32)


@functools.partial(jax.jit, static_argnames=("block_m", "interpret"))
def _run(x, weight, block_m=2048, interpret=False):
    n = x.shape[0]
    grid = (n // block_m,)
    out_w_t, out_i_t = pl.pallas_call(
        _router_block,
        grid=grid,
        in_specs=[
            pl.BlockSpec((block_m, DIM), lambda i: (i, 0)),
            pl.BlockSpec((N_EXPERTS, DIM), lambda i: (0, 0)),
        ],
        out_specs=[
            pl.BlockSpec((TOP_K, block_m), lambda i: (0, i)),
            pl.BlockSpec((TOP_K, block_m), lambda i: (0, i)),
        ],
        out_shape=[
            jax.ShapeDtypeStruct((TOP_K, n), jnp.float32),
            jax.ShapeDtypeStruct((TOP_K, n), jnp.int32),
        ],
        interpret=interpret,
    )(x, weight)
    return out_w_t.T, out_i_t.T


def kernel(x, weight):
    return tuple(_run(x, weight))
